# baseline (speedup 1.0000x reference)
import jax
import jax.numpy as jnp
from jax import lax
from jax.experimental import pallas as pl
from jax.experimental.pallas import tpu as pltpu
from jax.experimental.pallas import tpu_sc as plsc

NC = 2
NS = 16
NW = NC * NS

BATCH = 64
SEQ = 8192
DIM = 64
PAD = 128
CHUNK = SEQ // NW
NG = CHUNK // 16


def _sc_body(x_hbm, tab_hbm, pe_hbm, out_hbm,
             idx0, idx1, pe_v, buf0, buf1, tbuf0, tbuf1,
             gsem0, gsem1, ssem0, ssem1, isem0, isem1):
    wid = lax.axis_index("s") * NC + lax.axis_index("c")
    l0 = wid * CHUNK

    pltpu.sync_copy(pe_hbm.at[:, pl.ds(l0, CHUNK)], pe_v)

    idxs = (idx0, idx1)
    bufs = (buf0, buf1)
    tbufs = (tbuf0, tbuf1)
    gsems = (gsem0, gsem1)
    ssems = (ssem0, ssem1)
    isems = (isem0, isem1)

    for p in range(2):
        pltpu.sync_copy(x_hbm.at[p, pl.ds(l0, CHUNK)], idxs[p])
        pltpu.async_copy(tab_hbm.at[idxs[p]], bufs[p], gsems[p])

    iota16 = lax.iota(jnp.int32, 16)

    def step(b2, carry):
        for p in range(2):
            b = b2 * 2 + p
            buf, tbuf = bufs[p], tbufs[p]
            pltpu.make_async_copy(tab_hbm.at[idxs[p]], buf, gsems[p]).wait()

            @pl.when(b2 < BATCH // 2 - 1)
            def _():
                pltpu.async_copy(
                    x_hbm.at[b + 2, pl.ds(l0, CHUNK)], idxs[p], isems[p]
                )

            def d_body(d, c2):
                dvec = jnp.full((16,), d, dtype=jnp.int32)
                for g in range(NG):
                    lvec = g * 16 + iota16
                    vals = plsc.load_gather(buf, [lvec, dvec])
                    sl = pl.ds(g * 16, 16)
                    tbuf[d, sl] = vals + pe_v[d, sl]
                return c2

            lax.fori_loop(0, DIM, d_body, 0)
            pltpu.async_copy(tbuf, out_hbm.at[b, :, pl.ds(l0, CHUNK)], ssems[p])

        @pl.when(b2 < BATCH // 2 - 1)
        def _():
            for p in range(2):
                b = b2 * 2 + p
                pltpu.make_async_copy(
                    tbufs[p], out_hbm.at[b, :, pl.ds(l0, CHUNK)], ssems[p]
                ).wait()
                pltpu.make_async_copy(
                    x_hbm.at[b + 2, pl.ds(l0, CHUNK)], idxs[p], isems[p]
                ).wait()
                pltpu.async_copy(tab_hbm.at[idxs[p]], bufs[p], gsems[p])

        return carry

    lax.fori_loop(0, BATCH // 2, step, 0)

    for p in range(2):
        b = BATCH - 2 + p
        pltpu.make_async_copy(
            tbufs[p], out_hbm.at[b, :, pl.ds(l0, CHUNK)], ssems[p]
        ).wait()


@jax.jit
def kernel(x, embedding_table, positional_encoding):
    pe_t = positional_encoding.reshape(SEQ, DIM).T
    xi = x.astype(jnp.int32)
    tab_pad = jnp.pad(embedding_table, ((0, 0), (0, PAD - DIM)))

    mesh = plsc.VectorSubcoreMesh(
        core_axis_name="c", subcore_axis_name="s", num_cores=NC, num_subcores=NS
    )
    run = pl.kernel(
        _sc_body,
        out_type=jax.ShapeDtypeStruct((BATCH, DIM, SEQ), jnp.float32),
        mesh=mesh,
        scratch_types=[
            pltpu.VMEM((CHUNK,), jnp.int32),
            pltpu.VMEM((CHUNK,), jnp.int32),
            pltpu.VMEM((DIM, CHUNK), jnp.float32),
            pltpu.VMEM((CHUNK, PAD), jnp.float32),
            pltpu.VMEM((CHUNK, PAD), jnp.float32),
            pltpu.VMEM((DIM, CHUNK), jnp.float32),
            pltpu.VMEM((DIM, CHUNK), jnp.float32),
            pltpu.SemaphoreType.DMA,
            pltpu.SemaphoreType.DMA,
            pltpu.SemaphoreType.DMA,
            pltpu.SemaphoreType.DMA,
            pltpu.SemaphoreType.DMA,
            pltpu.SemaphoreType.DMA,
        ],
        compiler_params=pltpu.CompilerParams(needs_layout_passes=False),
    )
    out_t = run(xi, tab_pad, pe_t)
    return out_t.transpose(0, 2, 1)


# speedup vs baseline: 2.4509x; 2.4509x over previous
import jax
import jax.numpy as jnp
from jax import lax
from jax.experimental import pallas as pl
from jax.experimental.pallas import tpu as pltpu
from jax.experimental.pallas import tpu_sc as plsc

NC = 2
NS = 16
NW = NC * NS

BATCH = 64
SEQ = 8192
DIM = 64
PAD = 128
CHUNK = SEQ // NW
HALF = CHUNK // 2


def _sc_body(x_hbm, tab_hbm, pe_hbm, out_hbm,
             idx0, idx1, pe_v, buf0, buf1, sbuf0, sbuf1,
             gsem0, gsem1, ssem0, ssem1, isem0, isem1):
    wid = lax.axis_index("s") * NC + lax.axis_index("c")
    l0 = wid * CHUNK

    pltpu.sync_copy(pe_hbm.at[pl.ds(l0, CHUNK)], pe_v)

    idxs = (idx0, idx1)
    bufs = (buf0, buf1)
    sbufs = (sbuf0, sbuf1)
    gsems = (gsem0, gsem1)
    ssems = (ssem0, ssem1)
    isems = (isem0, isem1)

    for p in range(2):
        pltpu.sync_copy(x_hbm.at[0, pl.ds(l0 + p * HALF, HALF)], idxs[p])
        pltpu.async_copy(tab_hbm.at[idxs[p]], bufs[p], gsems[p])

    def step(b, carry):
        for p in range(2):
            off = p * HALF
            buf, sbuf = bufs[p], sbufs[p]
            pltpu.make_async_copy(tab_hbm.at[idxs[p]], buf, gsems[p]).wait()

            @pl.when(b < BATCH - 1)
            def _():
                pltpu.async_copy(
                    x_hbm.at[b + 1, pl.ds(l0 + off, HALF)], idxs[p], isems[p]
                )

            @pl.when(b > 0)
            def _():
                pltpu.make_async_copy(
                    sbuf, out_hbm.at[b - 1, pl.ds(l0 + off, HALF)], ssems[p]
                ).wait()

            def row_body(r, c2):
                for c in range(DIM // 16):
                    sl = pl.ds(c * 16, 16)
                    sbuf[r, sl] = buf[r, sl] + pe_v[r + off, sl]
                return c2

            lax.fori_loop(0, HALF, row_body, 0)
            pltpu.async_copy(sbuf, out_hbm.at[b, pl.ds(l0 + off, HALF)], ssems[p])

            @pl.when(b < BATCH - 1)
            def _():
                pltpu.make_async_copy(
                    x_hbm.at[b + 1, pl.ds(l0 + off, HALF)], idxs[p], isems[p]
                ).wait()
                pltpu.async_copy(tab_hbm.at[idxs[p]], buf, gsems[p])

        return carry

    lax.fori_loop(0, BATCH, step, 0)

    for p in range(2):
        off = p * HALF
        pltpu.make_async_copy(
            sbufs[p], out_hbm.at[BATCH - 1, pl.ds(l0 + off, HALF)], ssems[p]
        ).wait()


@jax.jit
def kernel(x, embedding_table, positional_encoding):
    pe2d = positional_encoding.reshape(SEQ, DIM)
    xi = x.astype(jnp.int32)
    tab_pad = jnp.pad(embedding_table, ((0, 0), (0, PAD - DIM)))

    mesh = plsc.VectorSubcoreMesh(
        core_axis_name="c", subcore_axis_name="s", num_cores=NC, num_subcores=NS
    )
    run = pl.kernel(
        _sc_body,
        out_type=jax.ShapeDtypeStruct((BATCH, SEQ, DIM), jnp.float32),
        mesh=mesh,
        scratch_types=[
            pltpu.VMEM((HALF,), jnp.int32),
            pltpu.VMEM((HALF,), jnp.int32),
            pltpu.VMEM((CHUNK, DIM), jnp.float32),
            pltpu.VMEM((HALF, PAD), jnp.float32),
            pltpu.VMEM((HALF, PAD), jnp.float32),
            pltpu.VMEM((HALF, DIM), jnp.float32),
            pltpu.VMEM((HALF, DIM), jnp.float32),
            pltpu.SemaphoreType.DMA,
            pltpu.SemaphoreType.DMA,
            pltpu.SemaphoreType.DMA,
            pltpu.SemaphoreType.DMA,
            pltpu.SemaphoreType.DMA,
            pltpu.SemaphoreType.DMA,
        ],
    )
    return run(xi, tab_pad, pe2d)
